# baseline (speedup 1.0000x reference)
import functools

import jax
import jax.numpy as jnp
from jax import lax
from jax.experimental import pallas as pl
from jax.experimental.pallas import tpu as pltpu
from jax.experimental.pallas import tpu_sc as plsc

N, E = 10000, 160000
IN_C, OUT_C, D_EDGE, HID = 32, 32, 16, 128
KK = IN_C * OUT_C

NC, NS = 2, 16
NW = NC * NS
CH = 128
NCHUNK = 40
EPW = NCHUNK * CH
E_PAD = NW * EPW
N_PAD = 10016
RPT = N_PAD // NS

BE = 2560
BN = 1000

@functools.lru_cache(maxsize=None)
def _sc_kernels(epad, gb):
    mesh = plsc.VectorSubcoreMesh(
        core_axis_name="c", subcore_axis_name="s",
        num_cores=NC, num_subcores=NS)

    epw = epad // NW
    nchunk = epw // CH
    assert nchunk % 2 == 0


    GB = gb
    GROWS = GB * CH
    NG = nchunk // GB
    assert NG % 2 == 0

    @functools.partial(
        pl.kernel,
        out_type=jax.ShapeDtypeStruct((epad, IN_C), jnp.float32),
        mesh=mesh,
        scratch_types=[
            pltpu.VMEM((nchunk, CH), jnp.int32),
            pltpu.VMEM((GROWS, IN_C), jnp.float32),
            pltpu.VMEM((GROWS, IN_C), jnp.float32),
            pltpu.SemaphoreType.DMA,
            pltpu.SemaphoreType.DMA,
            pltpu.SemaphoreType.DMA,
            pltpu.SemaphoreType.DMA,
        ],
        compiler_params=pltpu.CompilerParams(use_tc_tiling_on_sc=False),
    )
    def _gather_hsrc(h_hbm, idx_hbm, out_hbm, idx_v, bufa, bufb,
                     gsa, gsb, wsa, wsb):
        c = lax.axis_index("c")
        s = lax.axis_index("s")
        wid = s * NC + c
        base = wid * epw
        pltpu.sync_copy(idx_hbm.at[pl.ds(wid * nchunk, nchunk)], idx_v)

        def fire(g, buf, gsem):
            for q in range(GB):
                pltpu.async_copy(h_hbm.at[idx_v.at[g * GB + q]],
                                 buf.at[pl.ds(q * CH, CH)], gsem)

        def drain(buf, gsem):
            for q in range(GB):
                pltpu.make_async_copy(h_hbm.at[idx_v.at[q]],
                                      buf.at[pl.ds(q * CH, CH)], gsem).wait()

        def wb_wait(buf, wsem):
            pltpu.make_async_copy(buf, out_hbm.at[pl.ds(base, GROWS)],
                                  wsem).wait()

        fire(0, bufa, gsa)
        fire(1, bufb, gsb)

        @pl.loop(0, NG // 2)
        def _(t):
            g0 = 2 * t
            g1 = 2 * t + 1
            drain(bufa, gsa)
            pltpu.async_copy(bufa, out_hbm.at[pl.ds(base + g0 * GROWS, GROWS)],
                             wsa)
            drain(bufb, gsb)
            pltpu.async_copy(bufb, out_hbm.at[pl.ds(base + g1 * GROWS, GROWS)],
                             wsb)

            @pl.when(g0 + 2 < NG)
            def _():
                wb_wait(bufa, wsa)
                fire(g0 + 2, bufa, gsa)

            @pl.when(g1 + 2 < NG)
            def _():
                wb_wait(bufb, wsb)
                fire(g1 + 2, bufb, gsb)

        wb_wait(bufa, wsa)
        wb_wait(bufb, wsb)


    @functools.partial(
        pl.kernel,
        out_type=jax.ShapeDtypeStruct((NC, N_PAD, OUT_C), jnp.float32),
        mesh=mesh,
        scratch_types=[
            pltpu.VMEM((nchunk, CH), jnp.int32),
            pltpu.VMEM((CH, OUT_C), jnp.float32),
            pltpu.VMEM((CH, OUT_C), jnp.float32),
            pltpu.VMEM((RPT, OUT_C), jnp.float32),
            pltpu.VMEM_SHARED((N_PAD, OUT_C), jnp.float32),
            pltpu.SemaphoreType.DMA,
            pltpu.SemaphoreType.DMA,
        ],
        compiler_params=pltpu.CompilerParams(use_tc_tiling_on_sc=False),
    )
    def _scatter_msg(msg_hbm, dst_hbm, part_hbm, idx_v, rowsa, rowsb, zbuf,
                     shared, lsa, lsb):
        c = lax.axis_index("c")
        s = lax.axis_index("s")

        @pl.loop(0, RPT)
        def _(i):
            zbuf[i, pl.ds(0, 16)] = jnp.zeros((16,), jnp.float32)
            zbuf[i, pl.ds(16, 16)] = jnp.zeros((16,), jnp.float32)

        pltpu.sync_copy(zbuf, shared.at[pl.ds(s * RPT, RPT)])
        plsc.subcore_barrier()

        row0 = (c * NS + s) * nchunk
        pltpu.sync_copy(dst_hbm.at[pl.ds(row0, nchunk)], idx_v)
        base = (c * NS + s) * epw

        def load(j, rows, lsem):
            pltpu.async_copy(msg_hbm.at[pl.ds(base + j * CH, CH)], rows, lsem)

        def load_wait(rows, lsem):
            pltpu.make_async_copy(msg_hbm.at[pl.ds(base, CH)], rows,
                                  lsem).wait()

        load(0, rowsa, lsa)
        load(1, rowsb, lsb)

        @pl.loop(0, nchunk // 2)
        def _(t):
            j0 = 2 * t
            j1 = 2 * t + 1
            load_wait(rowsa, lsa)
            pltpu.sync_copy(rowsa, shared.at[idx_v.at[j0]], add=True)

            @pl.when(j0 + 2 < nchunk)
            def _():
                load(j0 + 2, rowsa, lsa)

            load_wait(rowsb, lsb)
            pltpu.sync_copy(rowsb, shared.at[idx_v.at[j1]], add=True)

            @pl.when(j1 + 2 < nchunk)
            def _():
                load(j1 + 2, rowsb, lsb)

        plsc.subcore_barrier()
        pltpu.sync_copy(shared.at[pl.ds(s * RPT, RPT)], zbuf)
        pltpu.sync_copy(zbuf, part_hbm.at[c, pl.ds(s * RPT, RPT)])

    return _gather_hsrc, _scatter_msg


def _msg_body(e_ref, hs_ref, W1_ref, b1_ref, W2_ref, b2_ref, R_ref, out_ref):
    hid = jnp.maximum(
        jnp.dot(e_ref[...], W1_ref[...], preferred_element_type=jnp.float32)
        + b1_ref[...], 0.0)
    Y = jnp.dot(hid.astype(jnp.bfloat16), W2_ref[...],
                preferred_element_type=jnp.float32) + b2_ref[...]
    hsb = hs_ref[...].astype(jnp.bfloat16)
    hrep = jnp.dot(hsb, R_ref[...], preferred_element_type=jnp.float32)
    p = [Y[:, 128 * g:128 * (g + 1)] * hrep[:, 128 * g:128 * (g + 1)]
         for g in range(8)]
    q = [p[0] + p[1], p[2] + p[3], p[4] + p[5], p[6] + p[7]]
    acc = (q[0] + q[1]) + (q[2] + q[3])
    out_ref[...] = ((acc[:, 0:32] + acc[:, 32:64])
                    + (acc[:, 64:96] + acc[:, 96:128]))


@functools.lru_cache(maxsize=None)
def _msg_call_for(epad):
    return pl.pallas_call(
        _msg_body,
        grid=(epad // BE,),
        in_specs=[
            pl.BlockSpec((BE, D_EDGE), lambda i: (i, 0)),
            pl.BlockSpec((BE, IN_C), lambda i: (i, 0)),
            pl.BlockSpec((D_EDGE, HID), lambda i: (0, 0)),
            pl.BlockSpec((1, HID), lambda i: (0, 0)),
            pl.BlockSpec((HID, KK), lambda i: (0, 0)),
            pl.BlockSpec((1, KK), lambda i: (0, 0)),
            pl.BlockSpec((IN_C, KK), lambda i: (0, 0)),
        ],
        out_specs=pl.BlockSpec((BE, OUT_C), lambda i: (i, 0)),
        out_shape=jax.ShapeDtypeStruct((epad, OUT_C), jnp.float32),
    )


def _final_body(h_ref, root_ref, bias_ref, p0_ref, p1_ref, p2_ref, p3_ref,
                out_ref):
    out_ref[...] = (
        jnp.dot(h_ref[...], root_ref[...], preferred_element_type=jnp.float32)
        + bias_ref[...] + (p0_ref[...] + p1_ref[...])
        + (p2_ref[...] + p3_ref[...]))


_final_call = pl.pallas_call(
    _final_body,
    grid=(N // BN,),
    in_specs=[
        pl.BlockSpec((BN, IN_C), lambda i: (i, 0)),
        pl.BlockSpec((IN_C, OUT_C), lambda i: (0, 0)),
        pl.BlockSpec((1, OUT_C), lambda i: (0, 0)),
        pl.BlockSpec((BN, OUT_C), lambda i: (i, 0)),
        pl.BlockSpec((BN, OUT_C), lambda i: (i, 0)),
        pl.BlockSpec((BN, OUT_C), lambda i: (i, 0)),
        pl.BlockSpec((BN, OUT_C), lambda i: (i, 0)),
    ],
    out_specs=pl.BlockSpec((BN, OUT_C), lambda i: (i, 0)),
    out_shape=jax.ShapeDtypeStruct((N, OUT_C), jnp.float32),
)

NSTRIPE = 2
E_S = E_PAD // NSTRIPE


def kernel(h, e, edge_index, W1, b1, W2, b2, root, bias):
    src = edge_index[0]
    dst = edge_index[1]
    pad = E_PAD - E
    src_p = jnp.concatenate(
        [src, jnp.zeros((pad,), jnp.int32)]).reshape(E_PAD // CH, CH)
    dst_p = jnp.concatenate(
        [dst, jnp.full((pad,), N, jnp.int32)]).reshape(E_PAD // CH, CH)
    e_p = jnp.concatenate([e, jnp.zeros((pad, D_EDGE), jnp.float32)], axis=0)

    _gather_hsrc, _scatter_msg = _sc_kernels(E_S, 2)
    _msg = _msg_call_for(E_S)
    R = jnp.repeat(jnp.eye(IN_C, dtype=jnp.bfloat16), OUT_C, axis=1)
    W2b = W2.astype(jnp.bfloat16)
    b1r = b1.reshape(1, HID)
    b2r = b2.reshape(1, KK)
    RC = E_S // CH

    parts = []
    for k in range(NSTRIPE):
        hsrc_k = _gather_hsrc(h, src_p[k * RC:(k + 1) * RC])
        msg_k = _msg(e_p[k * E_S:(k + 1) * E_S], hsrc_k, W1, b1r, W2b, b2r, R)
        part_k = _scatter_msg(msg_k, dst_p[k * RC:(k + 1) * RC])
        parts += [part_k[0, :N], part_k[1, :N]]

    return _final_call(h, root, bias.reshape(1, OUT_C), *parts)


# speedup vs baseline: 1.0004x; 1.0004x over previous
import functools

import jax
import jax.numpy as jnp
from jax import lax
from jax.experimental import pallas as pl
from jax.experimental.pallas import tpu as pltpu
from jax.experimental.pallas import tpu_sc as plsc

N, E = 10000, 160000
IN_C, OUT_C, D_EDGE, HID = 32, 32, 16, 128
KK = IN_C * OUT_C

NC, NS = 2, 16
NW = NC * NS
CH = 128
NCHUNK = 40
EPW = NCHUNK * CH
E_PAD = NW * EPW
N_PAD = 10016
RPT = N_PAD // NS

BE = 2560
BN = 1000

@functools.lru_cache(maxsize=None)
def _sc_kernels(epad, gb):
    mesh = plsc.VectorSubcoreMesh(
        core_axis_name="c", subcore_axis_name="s",
        num_cores=NC, num_subcores=NS)

    epw = epad // NW
    nchunk = epw // CH
    assert nchunk % 2 == 0


    GB = gb
    GROWS = GB * CH
    NG = nchunk // GB
    assert NG % 2 == 0

    @functools.partial(
        pl.kernel,
        out_type=jax.ShapeDtypeStruct((epad, IN_C), jnp.float32),
        mesh=mesh,
        scratch_types=[
            pltpu.VMEM((nchunk, CH), jnp.int32),
            pltpu.VMEM((GROWS, IN_C), jnp.float32),
            pltpu.VMEM((GROWS, IN_C), jnp.float32),
            pltpu.SemaphoreType.DMA,
            pltpu.SemaphoreType.DMA,
            pltpu.SemaphoreType.DMA,
            pltpu.SemaphoreType.DMA,
        ],
        compiler_params=pltpu.CompilerParams(use_tc_tiling_on_sc=False),
    )
    def _gather_hsrc(h_hbm, idx_hbm, out_hbm, idx_v, bufa, bufb,
                     gsa, gsb, wsa, wsb):
        c = lax.axis_index("c")
        s = lax.axis_index("s")
        wid = s * NC + c
        base = wid * epw
        pltpu.sync_copy(idx_hbm.at[pl.ds(wid * nchunk, nchunk)], idx_v)

        def fire(g, buf, gsem):
            for q in range(GB):
                pltpu.async_copy(h_hbm.at[idx_v.at[g * GB + q]],
                                 buf.at[pl.ds(q * CH, CH)], gsem)

        def drain(buf, gsem):
            for q in range(GB):
                pltpu.make_async_copy(h_hbm.at[idx_v.at[q]],
                                      buf.at[pl.ds(q * CH, CH)], gsem).wait()

        def wb_wait(buf, wsem):
            pltpu.make_async_copy(buf, out_hbm.at[pl.ds(base, GROWS)],
                                  wsem).wait()

        fire(0, bufa, gsa)
        fire(1, bufb, gsb)

        @pl.loop(0, NG // 2)
        def _(t):
            g0 = 2 * t
            g1 = 2 * t + 1
            drain(bufa, gsa)
            pltpu.async_copy(bufa, out_hbm.at[pl.ds(base + g0 * GROWS, GROWS)],
                             wsa)
            drain(bufb, gsb)
            pltpu.async_copy(bufb, out_hbm.at[pl.ds(base + g1 * GROWS, GROWS)],
                             wsb)

            @pl.when(g0 + 2 < NG)
            def _():
                wb_wait(bufa, wsa)
                fire(g0 + 2, bufa, gsa)

            @pl.when(g1 + 2 < NG)
            def _():
                wb_wait(bufb, wsb)
                fire(g1 + 2, bufb, gsb)

        wb_wait(bufa, wsa)
        wb_wait(bufb, wsb)


    @functools.partial(
        pl.kernel,
        out_type=jax.ShapeDtypeStruct((NC, N_PAD, OUT_C), jnp.float32),
        mesh=mesh,
        scratch_types=[
            pltpu.VMEM((nchunk, CH), jnp.int32),
            pltpu.VMEM((CH, OUT_C), jnp.float32),
            pltpu.VMEM((CH, OUT_C), jnp.float32),
            pltpu.VMEM((RPT, OUT_C), jnp.float32),
            pltpu.VMEM_SHARED((N_PAD, OUT_C), jnp.float32),
            pltpu.SemaphoreType.DMA,
            pltpu.SemaphoreType.DMA,
        ],
        compiler_params=pltpu.CompilerParams(use_tc_tiling_on_sc=False),
    )
    def _scatter_msg(msg_hbm, dst_hbm, part_hbm, idx_v, rowsa, rowsb, zbuf,
                     shared, lsa, lsb):
        c = lax.axis_index("c")
        s = lax.axis_index("s")

        @pl.loop(0, RPT)
        def _(i):
            zbuf[i, pl.ds(0, 16)] = jnp.zeros((16,), jnp.float32)
            zbuf[i, pl.ds(16, 16)] = jnp.zeros((16,), jnp.float32)

        pltpu.sync_copy(zbuf, shared.at[pl.ds(s * RPT, RPT)])
        plsc.subcore_barrier()

        row0 = (c * NS + s) * nchunk
        pltpu.sync_copy(dst_hbm.at[pl.ds(row0, nchunk)], idx_v)
        base = (c * NS + s) * epw

        def load(j, rows, lsem):
            pltpu.async_copy(msg_hbm.at[pl.ds(base + j * CH, CH)], rows, lsem)

        def load_wait(rows, lsem):
            pltpu.make_async_copy(msg_hbm.at[pl.ds(base, CH)], rows,
                                  lsem).wait()

        load(0, rowsa, lsa)
        load(1, rowsb, lsb)

        @pl.loop(0, nchunk // 2)
        def _(t):
            j0 = 2 * t
            j1 = 2 * t + 1
            load_wait(rowsa, lsa)
            pltpu.sync_copy(rowsa, shared.at[idx_v.at[j0]], add=True)

            @pl.when(j0 + 2 < nchunk)
            def _():
                load(j0 + 2, rowsa, lsa)

            load_wait(rowsb, lsb)
            pltpu.sync_copy(rowsb, shared.at[idx_v.at[j1]], add=True)

            @pl.when(j1 + 2 < nchunk)
            def _():
                load(j1 + 2, rowsb, lsb)

        plsc.subcore_barrier()
        pltpu.sync_copy(shared.at[pl.ds(s * RPT, RPT)], zbuf)
        pltpu.sync_copy(zbuf, part_hbm.at[c, pl.ds(s * RPT, RPT)])

    return _gather_hsrc, _scatter_msg


def _msg_body(e_ref, hs_ref, W1_ref, b1_ref, W2_ref, b2_ref, R_ref, out_ref):
    hid = jnp.maximum(
        jnp.dot(e_ref[...], W1_ref[...], preferred_element_type=jnp.float32)
        + b1_ref[...], 0.0)
    Y = jnp.dot(hid.astype(jnp.bfloat16), W2_ref[...],
                preferred_element_type=jnp.float32) + b2_ref[...]
    hsb = hs_ref[...].astype(jnp.bfloat16)
    hrep = jnp.dot(hsb, R_ref[...], preferred_element_type=jnp.float32)
    p = [Y[:, 128 * g:128 * (g + 1)] * hrep[:, 128 * g:128 * (g + 1)]
         for g in range(8)]
    q = [p[0] + p[1], p[2] + p[3], p[4] + p[5], p[6] + p[7]]
    acc = (q[0] + q[1]) + (q[2] + q[3])
    out_ref[...] = ((acc[:, 0:32] + acc[:, 32:64])
                    + (acc[:, 64:96] + acc[:, 96:128]))


@functools.lru_cache(maxsize=None)
def _msg_call_for(epad):
    return pl.pallas_call(
        _msg_body,
        grid=(epad // BE,),
        in_specs=[
            pl.BlockSpec((BE, D_EDGE), lambda i: (i, 0)),
            pl.BlockSpec((BE, IN_C), lambda i: (i, 0)),
            pl.BlockSpec((D_EDGE, HID), lambda i: (0, 0)),
            pl.BlockSpec((1, HID), lambda i: (0, 0)),
            pl.BlockSpec((HID, KK), lambda i: (0, 0)),
            pl.BlockSpec((1, KK), lambda i: (0, 0)),
            pl.BlockSpec((IN_C, KK), lambda i: (0, 0)),
        ],
        out_specs=pl.BlockSpec((BE, OUT_C), lambda i: (i, 0)),
        out_shape=jax.ShapeDtypeStruct((epad, OUT_C), jnp.float32),
    )


def _final_body(h_ref, root_ref, bias_ref, p0_ref, p1_ref, p2_ref, p3_ref,
                out_ref):
    out_ref[...] = (
        jnp.dot(h_ref[...], root_ref[...], preferred_element_type=jnp.float32)
        + bias_ref[...] + (p0_ref[...] + p1_ref[...])
        + (p2_ref[...] + p3_ref[...]))


_final_call = pl.pallas_call(
    _final_body,
    grid=(N // BN,),
    in_specs=[
        pl.BlockSpec((BN, IN_C), lambda i: (i, 0)),
        pl.BlockSpec((IN_C, OUT_C), lambda i: (0, 0)),
        pl.BlockSpec((1, OUT_C), lambda i: (0, 0)),
        pl.BlockSpec((BN, OUT_C), lambda i: (i, 0)),
        pl.BlockSpec((BN, OUT_C), lambda i: (i, 0)),
        pl.BlockSpec((BN, OUT_C), lambda i: (i, 0)),
        pl.BlockSpec((BN, OUT_C), lambda i: (i, 0)),
    ],
    out_specs=pl.BlockSpec((BN, OUT_C), lambda i: (i, 0)),
    out_shape=jax.ShapeDtypeStruct((N, OUT_C), jnp.float32),
)

NSTRIPE = 2
E_S = E_PAD // NSTRIPE


def kernel(h, e, edge_index, W1, b1, W2, b2, root, bias):
    src = edge_index[0]
    dst = edge_index[1]
    pad = E_PAD - E
    src_p = jnp.concatenate(
        [src, jnp.zeros((pad,), jnp.int32)]).reshape(E_PAD // CH, CH)
    dst_p = jnp.concatenate(
        [dst, jnp.full((pad,), N, jnp.int32)]).reshape(E_PAD // CH, CH)
    e_p = jnp.concatenate([e, jnp.zeros((pad, D_EDGE), jnp.float32)], axis=0)

    _gather_hsrc, _scatter_msg = _sc_kernels(E_S, 2)
    _msg = _msg_call_for(E_S)
    R = jnp.repeat(jnp.eye(IN_C, dtype=jnp.bfloat16), OUT_C, axis=1)
    W2b = W2.astype(jnp.bfloat16)
    b1r = b1.reshape(1, HID)
    b2r = b2.reshape(1, KK)
    RC = E_S // CH

    hsrcs = [_gather_hsrc(h, src_p[k * RC:(k + 1) * RC])
             for k in range(NSTRIPE)]
    parts = []
    for k in range(NSTRIPE):
        msg_k = _msg(e_p[k * E_S:(k + 1) * E_S], hsrcs[k], W1, b1r, W2b, b2r,
                     R)
        part_k = _scatter_msg(msg_k, dst_p[k * RC:(k + 1) * RC])
        parts += [part_k[0, :N], part_k[1, :N]]

    return _final_call(h, root, bias.reshape(1, OUT_C), *parts)


# speedup vs baseline: 1.0255x; 1.0251x over previous
import functools

import jax
import jax.numpy as jnp
from jax import lax
from jax.experimental import pallas as pl
from jax.experimental.pallas import tpu as pltpu
from jax.experimental.pallas import tpu_sc as plsc

N, E = 10000, 160000
IN_C, OUT_C, D_EDGE, HID = 32, 32, 16, 128
KK = IN_C * OUT_C

NC, NS = 2, 16
NW = NC * NS
CH = 128
NCHUNK = 40
EPW = NCHUNK * CH
E_PAD = NW * EPW
N_PAD = 10016
RPT = N_PAD // NS

BE = 5120
BN = 1000

@functools.lru_cache(maxsize=None)
def _sc_kernels():
    mesh = plsc.VectorSubcoreMesh(
        core_axis_name="c", subcore_axis_name="s",
        num_cores=NC, num_subcores=NS)


    GB = 4
    GROWS = GB * CH
    NG = NCHUNK // GB

    @functools.partial(
        pl.kernel,
        out_type=jax.ShapeDtypeStruct((E_PAD, IN_C), jnp.float32),
        mesh=mesh,
        scratch_types=[
            pltpu.VMEM((NCHUNK, CH), jnp.int32),
            pltpu.VMEM((GROWS, IN_C), jnp.float32),
            pltpu.VMEM((GROWS, IN_C), jnp.float32),
            pltpu.SemaphoreType.DMA,
            pltpu.SemaphoreType.DMA,
            pltpu.SemaphoreType.DMA,
            pltpu.SemaphoreType.DMA,
        ],
        compiler_params=pltpu.CompilerParams(use_tc_tiling_on_sc=False),
    )
    def _gather_hsrc(h_hbm, idx_hbm, out_hbm, idx_v, bufa, bufb,
                     gsa, gsb, wsa, wsb):
        c = lax.axis_index("c")
        s = lax.axis_index("s")
        wid = s * NC + c
        base = wid * EPW
        pltpu.sync_copy(idx_hbm.at[pl.ds(wid * NCHUNK, NCHUNK)], idx_v)

        def fire(g, buf, gsem):
            for q in range(GB):
                pltpu.async_copy(h_hbm.at[idx_v.at[g * GB + q]],
                                 buf.at[pl.ds(q * CH, CH)], gsem)

        def drain(buf, gsem):
            for q in range(GB):
                pltpu.make_async_copy(h_hbm.at[idx_v.at[q]],
                                      buf.at[pl.ds(q * CH, CH)], gsem).wait()

        def wb_wait(buf, wsem):
            pltpu.make_async_copy(buf, out_hbm.at[pl.ds(base, GROWS)],
                                  wsem).wait()

        fire(0, bufa, gsa)
        fire(1, bufb, gsb)

        @pl.loop(0, NG // 2)
        def _(t):
            g0 = 2 * t
            g1 = 2 * t + 1
            drain(bufa, gsa)
            pltpu.async_copy(bufa, out_hbm.at[pl.ds(base + g0 * GROWS, GROWS)],
                             wsa)
            drain(bufb, gsb)
            pltpu.async_copy(bufb, out_hbm.at[pl.ds(base + g1 * GROWS, GROWS)],
                             wsb)

            @pl.when(g0 + 2 < NG)
            def _():
                wb_wait(bufa, wsa)
                fire(g0 + 2, bufa, gsa)

            @pl.when(g1 + 2 < NG)
            def _():
                wb_wait(bufb, wsb)
                fire(g1 + 2, bufb, gsb)

        wb_wait(bufa, wsa)
        wb_wait(bufb, wsb)


    @functools.partial(
        pl.kernel,
        out_type=jax.ShapeDtypeStruct((NC, N_PAD, OUT_C), jnp.float32),
        mesh=mesh,
        scratch_types=[
            pltpu.VMEM((NCHUNK, CH), jnp.int32),
            pltpu.VMEM((CH, OUT_C), jnp.float32),
            pltpu.VMEM((CH, OUT_C), jnp.float32),
            pltpu.VMEM((RPT, OUT_C), jnp.float32),
            pltpu.VMEM_SHARED((N_PAD, OUT_C), jnp.float32),
            pltpu.SemaphoreType.DMA,
            pltpu.SemaphoreType.DMA,
        ],
        compiler_params=pltpu.CompilerParams(use_tc_tiling_on_sc=False),
    )
    def _scatter_msg(msg_hbm, dst_hbm, part_hbm, idx_v, rowsa, rowsb, zbuf,
                     shared, lsa, lsb):
        c = lax.axis_index("c")
        s = lax.axis_index("s")

        @pl.loop(0, RPT)
        def _(i):
            zbuf[i, pl.ds(0, 16)] = jnp.zeros((16,), jnp.float32)
            zbuf[i, pl.ds(16, 16)] = jnp.zeros((16,), jnp.float32)

        pltpu.sync_copy(zbuf, shared.at[pl.ds(s * RPT, RPT)])
        plsc.subcore_barrier()

        row0 = (c * NS + s) * NCHUNK
        pltpu.sync_copy(dst_hbm.at[pl.ds(row0, NCHUNK)], idx_v)
        base = (c * NS + s) * EPW

        def load(j, rows, lsem):
            pltpu.async_copy(msg_hbm.at[pl.ds(base + j * CH, CH)], rows, lsem)

        def load_wait(rows, lsem):
            pltpu.make_async_copy(msg_hbm.at[pl.ds(base, CH)], rows,
                                  lsem).wait()

        load(0, rowsa, lsa)
        load(1, rowsb, lsb)

        @pl.loop(0, NCHUNK // 2)
        def _(t):
            j0 = 2 * t
            j1 = 2 * t + 1
            load_wait(rowsa, lsa)
            pltpu.sync_copy(rowsa, shared.at[idx_v.at[j0]], add=True)

            @pl.when(j0 + 2 < NCHUNK)
            def _():
                load(j0 + 2, rowsa, lsa)

            load_wait(rowsb, lsb)
            pltpu.sync_copy(rowsb, shared.at[idx_v.at[j1]], add=True)

            @pl.when(j1 + 2 < NCHUNK)
            def _():
                load(j1 + 2, rowsb, lsb)

        plsc.subcore_barrier()
        pltpu.sync_copy(shared.at[pl.ds(s * RPT, RPT)], zbuf)
        pltpu.sync_copy(zbuf, part_hbm.at[c, pl.ds(s * RPT, RPT)])

    return _gather_hsrc, _scatter_msg


def _msg_body(e_ref, hs_ref, W1_ref, b1_ref, W2_ref, b2_ref, R_ref, out_ref):
    hid = jnp.maximum(
        jnp.dot(e_ref[...], W1_ref[...], preferred_element_type=jnp.float32)
        + b1_ref[...], 0.0)
    Y = jnp.dot(hid.astype(jnp.bfloat16), W2_ref[...],
                preferred_element_type=jnp.float32) + b2_ref[...]
    hsb = hs_ref[...].astype(jnp.bfloat16)
    hrep = jnp.dot(hsb, R_ref[...], preferred_element_type=jnp.float32)
    p = [Y[:, 128 * g:128 * (g + 1)] * hrep[:, 128 * g:128 * (g + 1)]
         for g in range(8)]
    q = [p[0] + p[1], p[2] + p[3], p[4] + p[5], p[6] + p[7]]
    acc = (q[0] + q[1]) + (q[2] + q[3])
    out_ref[...] = ((acc[:, 0:32] + acc[:, 32:64])
                    + (acc[:, 64:96] + acc[:, 96:128]))


_msg_call = pl.pallas_call(
    _msg_body,
    grid=(E_PAD // BE,),
    in_specs=[
        pl.BlockSpec((BE, D_EDGE), lambda i: (i, 0)),
        pl.BlockSpec((BE, IN_C), lambda i: (i, 0)),
        pl.BlockSpec((D_EDGE, HID), lambda i: (0, 0)),
        pl.BlockSpec((1, HID), lambda i: (0, 0)),
        pl.BlockSpec((HID, KK), lambda i: (0, 0)),
        pl.BlockSpec((1, KK), lambda i: (0, 0)),
        pl.BlockSpec((IN_C, KK), lambda i: (0, 0)),
    ],
    out_specs=pl.BlockSpec((BE, OUT_C), lambda i: (i, 0)),
    out_shape=jax.ShapeDtypeStruct((E_PAD, OUT_C), jnp.float32),
)


def _final_body(h_ref, root_ref, bias_ref, p0_ref, p1_ref, out_ref):
    out_ref[...] = (
        jnp.dot(h_ref[...], root_ref[...], preferred_element_type=jnp.float32)
        + bias_ref[...] + p0_ref[...] + p1_ref[...])


_final_call = pl.pallas_call(
    _final_body,
    grid=(N // BN,),
    in_specs=[
        pl.BlockSpec((BN, IN_C), lambda i: (i, 0)),
        pl.BlockSpec((IN_C, OUT_C), lambda i: (0, 0)),
        pl.BlockSpec((1, OUT_C), lambda i: (0, 0)),
        pl.BlockSpec((BN, OUT_C), lambda i: (i, 0)),
        pl.BlockSpec((BN, OUT_C), lambda i: (i, 0)),
    ],
    out_specs=pl.BlockSpec((BN, OUT_C), lambda i: (i, 0)),
    out_shape=jax.ShapeDtypeStruct((N, OUT_C), jnp.float32),
)


def kernel(h, e, edge_index, W1, b1, W2, b2, root, bias):
    src = edge_index[0]
    dst = edge_index[1]
    pad = E_PAD - E
    src_p = jnp.concatenate(
        [src, jnp.zeros((pad,), jnp.int32)]).reshape(E_PAD // CH, CH)
    dst_p = jnp.concatenate(
        [dst, jnp.full((pad,), N, jnp.int32)]).reshape(E_PAD // CH, CH)
    e_p = jnp.concatenate([e, jnp.zeros((pad, D_EDGE), jnp.float32)], axis=0)

    _gather_hsrc, _scatter_msg = _sc_kernels()
    hsrc = _gather_hsrc(h, src_p)
    R = jnp.repeat(jnp.eye(IN_C, dtype=jnp.bfloat16), OUT_C, axis=1)
    msg = _msg_call(e_p, hsrc, W1, b1.reshape(1, HID),
                    W2.astype(jnp.bfloat16), b2.reshape(1, KK), R)
    part = _scatter_msg(msg, dst_p)
    return _final_call(h, root, bias.reshape(1, OUT_C), part[0, :N], part[1, :N])
